# baseline (speedup 1.0000x reference)
import functools

import jax
import jax.numpy as jnp
from jax import lax
from jax.experimental import pallas as pl
from jax.experimental.pallas import tpu as pltpu
from jax.experimental.pallas import tpu_sc as plsc

E = 8
TOP_K = 2
H = 2048
M = 2048
T = 2048
EPAD = 128
BT = 128
ROWS_PAD = T * TOP_K + E * BT
NUM_TILES = ROWS_PAD // BT
NW = 32
TOK_PER_W = T // NW


def _plan_body(x_ref, rw_ref, wb0_ref, wb1_ref, pairs_i_ref, plan_ref):
    f32 = jnp.float32
    logits = jnp.dot(x_ref[...], rw_ref[...], preferred_element_type=f32)
    col = lax.broadcasted_iota(jnp.int32, (T, EPAD), 1)
    neg = f32(-1e30)
    l = jnp.where(col < E, logits, neg)
    m1 = jnp.max(l, axis=1, keepdims=True)
    i1 = jnp.min(jnp.where(l == m1, col, EPAD), axis=1, keepdims=True)
    l2 = jnp.where(col == i1, neg, l)
    m2 = jnp.max(l2, axis=1, keepdims=True)
    i2 = jnp.min(jnp.where(l2 == m2, col, EPAD), axis=1, keepdims=True)
    e21 = jnp.exp(m2 - m1)
    w1 = 1.0 / (1.0 + e21)
    w2 = e21 / (1.0 + e21)

    oh1 = (col == i1).astype(f32)
    oh2 = (col == i2).astype(f32)

    rI = lax.broadcasted_iota(jnp.int32, (128, 128), 0)
    cI = lax.broadcasted_iota(jnp.int32, (128, 128), 1)
    l_strict = (cI < rI).astype(f32)

    def excl_cumsum(oh):
        carry = jnp.zeros((1, EPAD), f32)
        outs = []
        for b in range(T // 128):
            blk = oh[b * 128:(b + 1) * 128, :]
            outs.append(jnp.dot(l_strict, blk, preferred_element_type=f32) + carry)
            carry = carry + jnp.sum(blk, axis=0, keepdims=True)
        return jnp.concatenate(outs, axis=0), carry

    excl1, counts1 = excl_cumsum(oh1)
    excl2, counts2 = excl_cumsum(oh2)
    counts = counts1 + counts2
    pc = jnp.ceil(counts / BT) * BT
    u_strict = (rI < cI).astype(f32)
    po = jnp.dot(pc, u_strict, preferred_element_type=f32)
    ends = po + pc

    rank1 = jnp.sum(excl1 * oh1, axis=1, keepdims=True)
    rank2 = jnp.sum(excl2 * oh2, axis=1, keepdims=True)
    po1 = jnp.sum(oh1 * po, axis=1, keepdims=True)
    po2 = jnp.sum(oh2 * po, axis=1, keepdims=True)
    c1sel = jnp.sum(oh2 * counts1, axis=1, keepdims=True)
    dst0 = po1 + rank1
    dst1 = po2 + c1sel + rank2

    lane = lax.broadcasted_iota(jnp.int32, (1, EPAD), 1)
    lane_f = lane.astype(f32)
    na = jnp.sum(pc) / BT
    ends_b = jnp.broadcast_to(ends, (128, 128))
    ends_col = jnp.sum(jnp.where(rI == cI, ends_b, 0.0), axis=1, keepdims=True)
    ind = jnp.where((cI.astype(f32) * BT >= ends_col) & (rI < E), 1.0, 0.0)
    te = jnp.sum(ind, axis=0, keepdims=True)
    pc_pos = (pc > 0) & (lane < E)
    la = jnp.max(jnp.where(pc_pos, lane_f, 0.0))
    te = jnp.where(lane_f < na, te, la)
    tr = jnp.minimum(lane_f, na - 1.0)
    na_row = jnp.full((1, EPAD), na, f32)

    shift = jnp.where(rI == (cI - 1), 1.0, 0.0)
    te_shift = jnp.dot(te, shift, preferred_element_type=f32)
    chg = jnp.where((te != te_shift) & (lane >= 1), 1.0, 0.0)
    u_incl = (rI <= cI).astype(f32)
    run = jnp.dot(chg, u_incl, preferred_element_type=f32)
    par = run - 2.0 * jnp.floor(run * 0.5)
    first = jnp.where((chg > 0) | (lane == 0), 1.0, 0.0)
    tot_chg = jnp.sum(chg)
    has_next = jnp.where(run < tot_chg, 1.0, 0.0)
    run_b = jnp.broadcast_to(run, (128, 128))
    run_col = jnp.sum(jnp.where(rI == cI, run_b, 0.0), axis=1, keepdims=True)
    te_b = jnp.broadcast_to(te, (128, 128))
    te_col = jnp.sum(jnp.where(rI == cI, te_b, 0.0), axis=1, keepdims=True)
    mnext = run_col == (run + 1.0)
    ne = jnp.max(jnp.where(mnext, jnp.broadcast_to(te_col, (128, 128)), -1.0),
                 axis=0, keepdims=True)
    ne = jnp.maximum(ne, 0.0)

    plan = jnp.concatenate(
        [te, tr, na_row, par, first, has_next, ne,
         jnp.zeros((1, EPAD), f32)], axis=0)
    plan_ref[...] = plan.astype(jnp.int32)

    c0 = col == 0
    c1 = col == 1
    wb0_ref[...] = jnp.broadcast_to(w1, (T, EPAD))
    wb1_ref[...] = jnp.broadcast_to(w2, (T, EPAD))
    pairs_i_ref[...] = (jnp.where(c0, dst0, 0.0)
                        + jnp.where(c1, dst1, 0.0)).astype(jnp.int32)


def _plan_call(hidden, rw_pad):
    return pl.pallas_call(
        _plan_body,
        out_shape=(
            jax.ShapeDtypeStruct((T, EPAD), jnp.float32),
            jax.ShapeDtypeStruct((T, EPAD), jnp.float32),
            jax.ShapeDtypeStruct((T, EPAD), jnp.int32),
            jax.ShapeDtypeStruct((8, EPAD), jnp.int32),
        ),
    )(hidden, rw_pad)


def _dispatch_call(hidden, dsts, wb0, wb1):
    mesh = plsc.VectorSubcoreMesh(core_axis_name="c", subcore_axis_name="s", num_cores=2, num_subcores=16)

    @functools.partial(
        pl.kernel,
        out_type=(
            jax.ShapeDtypeStruct((ROWS_PAD, H), jnp.float32),
            jax.ShapeDtypeStruct((ROWS_PAD, EPAD), jnp.float32),
        ),
        mesh=mesh,
        scratch_types=[
            pltpu.VMEM((4, 32), jnp.int32),
            pltpu.VMEM((32, H), jnp.float32),
            pltpu.VMEM((32, EPAD), jnp.float32),
            pltpu.SemaphoreType.DMA,
            pltpu.SemaphoreType.DMA,
        ],
    )
    def dispatch(hidden_hbm, dsts_hbm, wb0_hbm, wb1_hbm, xs_hbm, ws_hbm,
                 idx_v, rows_v, wrows_v, sem, semw):
        wid = lax.axis_index("s") * 2 + lax.axis_index("c")
        slot = wid // 16
        tok_base = (wid % 16) * 128
        for j in range(4):
            pltpu.sync_copy(dsts_hbm.at[slot, pl.ds(tok_base + j * 32, 32)],
                            idx_v.at[j])
        for j in range(4):
            pltpu.sync_copy(hidden_hbm.at[pl.ds(tok_base + j * 32, 32), :],
                            rows_v)

            @pl.when(slot == 0)
            def _():
                pltpu.sync_copy(
                    wb0_hbm.at[pl.ds(tok_base + j * 32, 32), :], wrows_v)

            @pl.when(slot == 1)
            def _():
                pltpu.sync_copy(
                    wb1_hbm.at[pl.ds(tok_base + j * 32, 32), :], wrows_v)

            cpw = pltpu.async_copy(wrows_v, ws_hbm.at[idx_v.at[j]], semw)
            pltpu.async_copy(rows_v, xs_hbm.at[idx_v.at[j]], sem).wait()
            cpw.wait()

    return dispatch(hidden, dsts, wb0, wb1)


def _gmm_body(p_ref, x_ref, w_any, *rest, gelu):
    if gelu:
        o_ref, wbuf, sem = rest
    else:
        ws_ref, o_ref, wbuf, sem = rest
    i = pl.program_id(0)
    par = p_ref[3, i]
    first = p_ref[4, i]

    @pl.when(i == 0)
    def _():
        pltpu.make_async_copy(w_any.at[p_ref[0, 0]], wbuf.at[0], sem).start()

    @pl.when(first == 1)
    def _():
        pltpu.make_async_copy(w_any.at[0], wbuf.at[par], sem).wait()

    @pl.when(jnp.logical_and(first == 1, p_ref[5, i] == 1))
    def _():
        pltpu.make_async_copy(w_any.at[p_ref[6, i]], wbuf.at[1 - par],
                              sem).start()

    @pl.when(i < p_ref[2, 0])
    def _():
        acc = jnp.dot(x_ref[...], wbuf[par],
                      preferred_element_type=jnp.float32)
        if gelu:
            o_ref[...] = jax.nn.gelu(acc)
        else:
            o_ref[...] = acc * ws_ref[:, :1]


def _gmm_call(plan, x, w, din, dout, gelu, ws=None):
    in_specs = [
        pl.BlockSpec((BT, din), lambda i, p: (p[1, i], 0)),
        pl.BlockSpec(memory_space=pl.ANY),
    ]
    args = (plan, x, w)
    if ws is not None:
        in_specs.append(pl.BlockSpec((BT, EPAD), lambda i, p: (p[1, i], 0)))
        args = (plan, x, w, ws)
    grid_spec = pltpu.PrefetchScalarGridSpec(
        num_scalar_prefetch=1,
        grid=(NUM_TILES,),
        in_specs=in_specs,
        out_specs=pl.BlockSpec((BT, dout), lambda i, p: (p[1, i], 0)),
        scratch_shapes=[
            pltpu.VMEM((2, din, dout), jnp.float32),
            pltpu.SemaphoreType.DMA,
        ],
    )
    return pl.pallas_call(
        functools.partial(_gmm_body, gelu=gelu),
        grid_spec=grid_spec,
        out_shape=jax.ShapeDtypeStruct((ROWS_PAD, dout), jnp.float32),
        compiler_params=pltpu.CompilerParams(
            dimension_semantics=("arbitrary",)),
    )(*args)


def _combine_call(y, dsts):
    mesh = plsc.VectorSubcoreMesh(core_axis_name="c", subcore_axis_name="s", num_cores=2, num_subcores=16)

    @functools.partial(
        pl.kernel,
        out_type=jax.ShapeDtypeStruct((T, H), jnp.float32),
        mesh=mesh,
        scratch_types=[
            pltpu.VMEM((4, 32), jnp.int32),
            pltpu.VMEM((32, H), jnp.float32),
            pltpu.VMEM((16, H), jnp.float32),
            pltpu.SemaphoreType.DMA,
        ],
    )
    def combine(y_hbm, dsts_hbm, out_hbm, idx_v, y01, ob, sem):
        wid = lax.axis_index("s") * 2 + lax.axis_index("c")
        base = wid * TOK_PER_W
        for c in range(4):
            pltpu.sync_copy(dsts_hbm.at[0, pl.ds(base + c * 16, 16)],
                            idx_v.at[c, pl.ds(0, 16)])
            pltpu.sync_copy(dsts_hbm.at[1, pl.ds(base + c * 16, 16)],
                            idx_v.at[c, pl.ds(16, 16)])
        for c in range(4):
            pltpu.async_copy(y_hbm.at[idx_v.at[c]], y01, sem).wait()
            for i in range(16):

                def body(v, carry, i=i):
                    sl = pl.ds(v * 16, 16)
                    ob[i, sl] = y01[i, sl] + y01[16 + i, sl]
                    return carry

                lax.fori_loop(0, H // 16, body, 0)
            pltpu.sync_copy(ob, out_hbm.at[pl.ds(base + c * 16, 16), :])

    return combine(y, dsts)


@jax.jit
def kernel(hidden_states, router_w, wi, wd):
    rw_pad = jnp.zeros((H, EPAD), jnp.float32).at[:, :E].set(router_w)
    wb0, wb1, pairs_i, plan = _plan_call(hidden_states, rw_pad)
    dsts = pairs_i[:, :TOP_K].T
    x_sorted, w_sorted = _dispatch_call(hidden_states, dsts, wb0, wb1)
    h = _gmm_call(plan, x_sorted, wi, H, M, gelu=True)
    y = _gmm_call(plan, h, wd, M, H, gelu=False, ws=w_sorted)
    return _combine_call(y, dsts)


# speedup vs baseline: 1.0654x; 1.0654x over previous
import functools

import jax
import jax.numpy as jnp
from jax import lax
from jax.experimental import pallas as pl
from jax.experimental.pallas import tpu as pltpu
from jax.experimental.pallas import tpu_sc as plsc

E = 8
TOP_K = 2
H = 2048
M = 2048
T = 2048
EPAD = 128
BT = 256
ROWS_PAD = T * TOP_K + E * BT
NUM_TILES = ROWS_PAD // BT
NW = 32
TOK_PER_W = T // NW


def _plan_body(x_ref, rw_ref, wb0_ref, wb1_ref, pairs_i_ref, plan_ref):
    f32 = jnp.float32
    logits = jnp.dot(x_ref[...], rw_ref[...], preferred_element_type=f32)
    col = lax.broadcasted_iota(jnp.int32, (T, EPAD), 1)
    neg = f32(-1e30)
    l = jnp.where(col < E, logits, neg)
    m1 = jnp.max(l, axis=1, keepdims=True)
    i1 = jnp.min(jnp.where(l == m1, col, EPAD), axis=1, keepdims=True)
    l2 = jnp.where(col == i1, neg, l)
    m2 = jnp.max(l2, axis=1, keepdims=True)
    i2 = jnp.min(jnp.where(l2 == m2, col, EPAD), axis=1, keepdims=True)
    e21 = jnp.exp(m2 - m1)
    w1 = 1.0 / (1.0 + e21)
    w2 = e21 / (1.0 + e21)

    oh1 = (col == i1).astype(f32)
    oh2 = (col == i2).astype(f32)

    rI = lax.broadcasted_iota(jnp.int32, (128, 128), 0)
    cI = lax.broadcasted_iota(jnp.int32, (128, 128), 1)
    l_strict = (cI < rI).astype(f32)

    def excl_cumsum(oh):
        carry = jnp.zeros((1, EPAD), f32)
        outs = []
        for b in range(T // 128):
            blk = oh[b * 128:(b + 1) * 128, :]
            outs.append(jnp.dot(l_strict, blk, preferred_element_type=f32) + carry)
            carry = carry + jnp.sum(blk, axis=0, keepdims=True)
        return jnp.concatenate(outs, axis=0), carry

    excl1, counts1 = excl_cumsum(oh1)
    excl2, counts2 = excl_cumsum(oh2)
    counts = counts1 + counts2
    pc = jnp.ceil(counts / BT) * BT
    u_strict = (rI < cI).astype(f32)
    po = jnp.dot(pc, u_strict, preferred_element_type=f32)
    ends = po + pc

    rank1 = jnp.sum(excl1 * oh1, axis=1, keepdims=True)
    rank2 = jnp.sum(excl2 * oh2, axis=1, keepdims=True)
    po1 = jnp.sum(oh1 * po, axis=1, keepdims=True)
    po2 = jnp.sum(oh2 * po, axis=1, keepdims=True)
    c1sel = jnp.sum(oh2 * counts1, axis=1, keepdims=True)
    dst0 = po1 + rank1
    dst1 = po2 + c1sel + rank2

    lane = lax.broadcasted_iota(jnp.int32, (1, EPAD), 1)
    lane_f = lane.astype(f32)
    na = jnp.sum(pc) / BT
    ends_b = jnp.broadcast_to(ends, (128, 128))
    ends_col = jnp.sum(jnp.where(rI == cI, ends_b, 0.0), axis=1, keepdims=True)
    ind = jnp.where((cI.astype(f32) * BT >= ends_col) & (rI < E), 1.0, 0.0)
    te = jnp.sum(ind, axis=0, keepdims=True)
    pc_pos = (pc > 0) & (lane < E)
    la = jnp.max(jnp.where(pc_pos, lane_f, 0.0))
    te = jnp.where(lane_f < na, te, la)
    tr = jnp.minimum(lane_f, na - 1.0)
    na_row = jnp.full((1, EPAD), na, f32)

    shift = jnp.where(rI == (cI - 1), 1.0, 0.0)
    te_shift = jnp.dot(te, shift, preferred_element_type=f32)
    chg = jnp.where((te != te_shift) & (lane >= 1), 1.0, 0.0)
    u_incl = (rI <= cI).astype(f32)
    run = jnp.dot(chg, u_incl, preferred_element_type=f32)
    par = run - 2.0 * jnp.floor(run * 0.5)
    first = jnp.where((chg > 0) | (lane == 0), 1.0, 0.0)
    tot_chg = jnp.sum(chg)
    has_next = jnp.where(run < tot_chg, 1.0, 0.0)
    run_b = jnp.broadcast_to(run, (128, 128))
    run_col = jnp.sum(jnp.where(rI == cI, run_b, 0.0), axis=1, keepdims=True)
    te_b = jnp.broadcast_to(te, (128, 128))
    te_col = jnp.sum(jnp.where(rI == cI, te_b, 0.0), axis=1, keepdims=True)
    mnext = run_col == (run + 1.0)
    ne = jnp.max(jnp.where(mnext, jnp.broadcast_to(te_col, (128, 128)), -1.0),
                 axis=0, keepdims=True)
    ne = jnp.maximum(ne, 0.0)

    plan = jnp.concatenate(
        [te, tr, na_row, par, first, has_next, ne,
         jnp.zeros((1, EPAD), f32)], axis=0)
    plan_ref[...] = plan.astype(jnp.int32)

    c0 = col == 0
    c1 = col == 1
    wb0_ref[...] = jnp.broadcast_to(w1, (T, EPAD))
    wb1_ref[...] = jnp.broadcast_to(w2, (T, EPAD))
    pairs_i_ref[...] = (jnp.where(c0, dst0, 0.0)
                        + jnp.where(c1, dst1, 0.0)).astype(jnp.int32)


def _plan_call(hidden, rw_pad):
    return pl.pallas_call(
        _plan_body,
        out_shape=(
            jax.ShapeDtypeStruct((T, EPAD), jnp.float32),
            jax.ShapeDtypeStruct((T, EPAD), jnp.float32),
            jax.ShapeDtypeStruct((T, EPAD), jnp.int32),
            jax.ShapeDtypeStruct((8, EPAD), jnp.int32),
        ),
    )(hidden, rw_pad)


def _dispatch_call(hidden, dsts):
    mesh = plsc.VectorSubcoreMesh(core_axis_name="c", subcore_axis_name="s", num_cores=2, num_subcores=16)

    @functools.partial(
        pl.kernel,
        out_type=jax.ShapeDtypeStruct((ROWS_PAD, H), jnp.float32),
        mesh=mesh,
        scratch_types=[
            pltpu.VMEM((4, 32), jnp.int32),
            pltpu.VMEM((32, H), jnp.float32),
            pltpu.SemaphoreType.DMA,
        ],
    )
    def dispatch(hidden_hbm, dsts_hbm, xs_hbm, idx_v, rows_v, sem):
        wid = lax.axis_index("s") * 2 + lax.axis_index("c")
        slot = wid // 16
        tok_base = (wid % 16) * 128
        for j in range(4):
            pltpu.sync_copy(dsts_hbm.at[slot, pl.ds(tok_base + j * 32, 32)],
                            idx_v.at[j])
        for j in range(4):
            pltpu.sync_copy(hidden_hbm.at[pl.ds(tok_base + j * 32, 32), :],
                            rows_v)
            pltpu.async_copy(rows_v, xs_hbm.at[idx_v.at[j]], sem).wait()

    return dispatch(hidden, dsts)


def _gmm_body(p_ref, x_ref, w_any, o_ref, wbuf, sem, *, gelu):
    i = pl.program_id(0)
    par = p_ref[3, i]
    first = p_ref[4, i]

    @pl.when(i == 0)
    def _():
        pltpu.make_async_copy(w_any.at[p_ref[0, 0]], wbuf.at[0], sem).start()

    @pl.when(first == 1)
    def _():
        pltpu.make_async_copy(w_any.at[0], wbuf.at[par], sem).wait()

    @pl.when(jnp.logical_and(first == 1, p_ref[5, i] == 1))
    def _():
        pltpu.make_async_copy(w_any.at[p_ref[6, i]], wbuf.at[1 - par],
                              sem).start()

    @pl.when(i < p_ref[2, 0])
    def _():
        acc = jnp.dot(x_ref[...], wbuf[par],
                      preferred_element_type=jnp.float32)
        o_ref[...] = jax.nn.gelu(acc) if gelu else acc


def _gmm_call(plan, x, w, din, dout, gelu):
    grid_spec = pltpu.PrefetchScalarGridSpec(
        num_scalar_prefetch=1,
        grid=(NUM_TILES,),
        in_specs=[
            pl.BlockSpec((BT, din), lambda i, p: (p[1, i], 0)),
            pl.BlockSpec(memory_space=pl.ANY),
        ],
        out_specs=pl.BlockSpec((BT, dout), lambda i, p: (p[1, i], 0)),
        scratch_shapes=[
            pltpu.VMEM((2, din, dout), jnp.float32),
            pltpu.SemaphoreType.DMA,
        ],
    )
    return pl.pallas_call(
        functools.partial(_gmm_body, gelu=gelu),
        grid_spec=grid_spec,
        out_shape=jax.ShapeDtypeStruct((ROWS_PAD, dout), jnp.float32),
        compiler_params=pltpu.CompilerParams(
            dimension_semantics=("arbitrary",)),
    )(plan, x, w)


def _combine_call(y, dsts, wb0, wb1):
    mesh = plsc.VectorSubcoreMesh(core_axis_name="c", subcore_axis_name="s", num_cores=2, num_subcores=16)

    @functools.partial(
        pl.kernel,
        out_type=jax.ShapeDtypeStruct((T, H), jnp.float32),
        mesh=mesh,
        scratch_types=[
            pltpu.VMEM((4, 16), jnp.int32),
            pltpu.VMEM((4, 16), jnp.int32),
            pltpu.VMEM((16, EPAD), jnp.float32),
            pltpu.VMEM((16, EPAD), jnp.float32),
            pltpu.VMEM((16, H), jnp.float32),
            pltpu.VMEM((16, H), jnp.float32),
            pltpu.VMEM((16, H), jnp.float32),
            pltpu.SemaphoreType.DMA,
            pltpu.SemaphoreType.DMA,
        ],
    )
    def combine(y_hbm, dsts_hbm, wb0_hbm, wb1_hbm, out_hbm,
                idx0, idx1, w0m, w1m, y0, y1, ob, sem0, sem1):
        wid = lax.axis_index("s") * 2 + lax.axis_index("c")
        base = wid * TOK_PER_W
        for c in range(4):
            pltpu.sync_copy(dsts_hbm.at[0, pl.ds(base + c * 16, 16)],
                            idx0.at[c])
            pltpu.sync_copy(dsts_hbm.at[1, pl.ds(base + c * 16, 16)],
                            idx1.at[c])
        for c in range(4):
            cp0 = pltpu.async_copy(y_hbm.at[idx0.at[c]], y0, sem0)
            cp1 = pltpu.async_copy(y_hbm.at[idx1.at[c]], y1, sem1)
            pltpu.sync_copy(wb0_hbm.at[pl.ds(base + c * 16, 16), :], w0m)
            pltpu.sync_copy(wb1_hbm.at[pl.ds(base + c * 16, 16), :], w1m)
            cp0.wait()
            cp1.wait()
            for i in range(16):
                w0b = w0m[i, pl.ds(0, 16)]
                w1b = w1m[i, pl.ds(0, 16)]

                def body(v, carry, i=i, w0b=w0b, w1b=w1b):
                    sl = pl.ds(v * 16, 16)
                    ob[i, sl] = w0b * y0[i, sl] + w1b * y1[i, sl]
                    return carry

                lax.fori_loop(0, H // 16, body, 0)
            pltpu.sync_copy(ob, out_hbm.at[pl.ds(base + c * 16, 16), :])

    return combine(y, dsts, wb0, wb1)


@jax.jit
def kernel(hidden_states, router_w, wi, wd):
    rw_pad = jnp.zeros((H, EPAD), jnp.float32).at[:, :E].set(router_w)
    wb0, wb1, pairs_i, plan = _plan_call(hidden_states, rw_pad)
    dsts = pairs_i[:, :TOP_K].T
    x_sorted = _dispatch_call(hidden_states, dsts)
    h = _gmm_call(plan, x_sorted, wi, H, M, gelu=True)
    y = _gmm_call(plan, h, wd, M, H, gelu=False)
    return _combine_call(y, dsts, wb0, wb1)
